# baseline (speedup 1.0000x reference)
import functools

import jax
import jax.numpy as jnp
from jax import lax
from jax.experimental import pallas as pl
from jax.experimental.pallas import tpu as pltpu
from jax.experimental.pallas import tpu_sc as plsc

N_CONCEPTS = 1000000
NLAYERS = 2
HIDDEN = 64
BATCH = 16384

_info = plsc.get_sparse_core_info()
NC = _info.num_cores
NS = _info.num_subcores
NW = NC * NS
B_PER_W = BATCH // NW
CHUNK = 128
NCHUNK = B_PER_W // CHUNK
NBUF = 3
NGATHER = 2 * NCHUNK
LANES = 16


def _sc_body(x_ref, c_ref, h_ref, c_out, h_out, idx_v, bufs, pads, gsems, wsems):
    wid = lax.axis_index("s") * NC + lax.axis_index("c")
    row0 = pl.multiple_of((wid // 2) * (2 * NCHUNK), 8)
    pltpu.sync_copy(x_ref.at[pl.ds(row0, 2 * NCHUNK), :], idx_v)

    tables = (c_ref, h_ref)
    outs = (c_out, h_out)

    def start_gather(g):
        t, j = divmod(g, NCHUNK)
        b = g % NBUF
        return pltpu.async_copy(
            tables[t].at[idx_v.at[(wid % 2) * NCHUNK + j]],
            bufs.at[b],
            gsems.at[b],
        )

    gh = [start_gather(g) for g in range(NBUF)]
    gh += [None] * (NGATHER - NBUF)
    wh = [None, None]

    for g in range(NGATHER):
        t, j = divmod(g, NCHUNK)
        b = g % NBUF
        p = g % 2
        gh[g].wait()
        if wh[p] is not None:
            for w in wh[p]:
                w.wait()
            wh[p] = None
        def repack(r, _):
            for k in range(NLAYERS * HIDDEN // LANES):
                v = bufs[b, r, pl.ds(k * LANES, LANES)]
                l, kk = divmod(k, HIDDEN // LANES)
                pads[p, l, r, pl.ds(kk * LANES, LANES)] = v
            return 0
        lax.fori_loop(0, CHUNK, repack, 0, unroll=2)
        base = wid * B_PER_W + j * CHUNK
        wh[p] = [
            pltpu.async_copy(
                pads.at[p, l],
                outs[t].at[l, pl.ds(base, CHUNK), :],
                wsems.at[p],
            )
            for l in range(NLAYERS)
        ]
        ng = g + NBUF
        if ng < NGATHER:
            gh[ng] = start_gather(ng)
    for ws in wh:
        if ws is not None:
            for w in ws:
                w.wait()


@functools.partial(jax.jit, static_argnums=())
def kernel(x, c_table, h_table):
    x2 = x.reshape(BATCH // CHUNK, CHUNK)
    out_sds = jax.ShapeDtypeStruct((NLAYERS, BATCH, HIDDEN), jnp.float32)
    run = pl.kernel(
        _sc_body,
        out_type=(out_sds, out_sds),
        mesh=plsc.VectorSubcoreMesh(core_axis_name="c", subcore_axis_name="s"),
        scratch_types=[
            pltpu.VMEM((2 * NCHUNK, CHUNK), jnp.int32),
            pltpu.VMEM((NBUF, CHUNK, NLAYERS * HIDDEN), jnp.float32),
            pltpu.VMEM((2, NLAYERS, CHUNK, HIDDEN), jnp.float32),
            pltpu.SemaphoreType.DMA((NBUF,)),
            pltpu.SemaphoreType.DMA((2,)),
        ],
    )
    c_init, h_init = run(x2, c_table, h_table)
    return (c_init, h_init)
